# Initial kernel scaffold; baseline (speedup 1.0000x reference)
import jax
import jax.numpy as jnp
from jax.experimental import pallas as pl


def kernel(item_seq, item_degree, node_table, degree_table):
    raise NotImplementedError("write your pallas kernel here")


# speedup vs baseline: 3.1374x; 3.1374x over previous
import functools

import jax
import jax.numpy as jnp
from jax import lax
from jax.experimental import pallas as pl
from jax.experimental.pallas import tpu as pltpu
from jax.experimental.pallas import tpu_sc as plsc

L = 16
NW = 32
CH = 128


def _make_sc_kernel(n_total, n_deg, d_model):
  assert n_total % (NW * CH) == 0
  n_per_w = n_total // NW
  n_chunks = n_per_w // CH
  mesh = plsc.VectorSubcoreMesh(core_axis_name="c", subcore_axis_name="s")

  @functools.partial(
      pl.kernel,
      out_type=jax.ShapeDtypeStruct((n_total, d_model), jnp.float32),
      mesh=mesh,
      scratch_types=[
          pltpu.VMEM((CH,), jnp.int32),
          pltpu.VMEM((CH,), jnp.int32),
          pltpu.VMEM((CH, d_model), jnp.float32),
          pltpu.VMEM((n_deg, d_model), jnp.float32),
          pltpu.VMEM((d_model,), jnp.float32),
          pltpu.VMEM_SHARED((2 * n_deg, d_model), jnp.float32),
          pltpu.SemaphoreType.DMA,
      ],
      compiler_params=pltpu.CompilerParams(use_tc_tiling_on_sc=False),
  )
  def body(nidx_hbm, didx_hbm, node_hbm, deg_hbm, out_hbm,
           nidx_v, didx_v, rows_v, dt_v, nt0_v, dt_sp, sem):
    c = lax.axis_index("c")
    s = lax.axis_index("s")
    wid = s * 2 + c
    base_w = wid * n_per_w

    @pl.when(s == 0)
    def _stage():
      pltpu.sync_copy(deg_hbm, dt_v)
      pltpu.sync_copy(node_hbm.at[0], nt0_v)
      for i in range(d_model // L):
        dt_v[0, pl.ds(i * L, L)] = jnp.zeros((L,), jnp.float32)
      pltpu.sync_copy(dt_v, dt_sp.at[pl.ds(0, n_deg)])

      def sub_row(r, carry):
        for i in range(d_model // L):
          sl = pl.ds(i * L, L)
          dt_v[r, sl] = dt_v[r, sl] - nt0_v[sl]
        return carry
      lax.fori_loop(0, n_deg, sub_row, 0)
      pltpu.sync_copy(dt_v, dt_sp.at[pl.ds(n_deg, n_deg)])
    plsc.subcore_barrier()

    def chunk(g, carry):
      base = base_w + g * CH
      pltpu.sync_copy(nidx_hbm.at[pl.ds(base, CH)], nidx_v)
      pltpu.sync_copy(didx_hbm.at[pl.ds(base, CH)], didx_v)
      for grp in range(CH // L):
        sl = pl.ds(grp * L, L)
        d = didx_v[sl]
        didx_v[sl] = jnp.where(nidx_v[sl] == 0, d + n_deg, d)
      pltpu.async_copy(node_hbm.at[nidx_v], rows_v, sem).wait()
      pltpu.async_copy(dt_sp.at[didx_v], rows_v, sem, add=True).wait()
      pltpu.sync_copy(rows_v, out_hbm.at[pl.ds(base, CH)])
      return carry

    lax.fori_loop(0, n_chunks, chunk, 0)

  return body


def kernel(item_seq, item_degree, node_table, degree_table):
  b, sq = item_seq.shape
  n_deg, d_model = degree_table.shape
  n_total = b * sq
  nidx = item_seq.reshape(n_total).astype(jnp.int32)
  didx = item_degree.reshape(n_total).astype(jnp.int32)
  out = _make_sc_kernel(n_total, n_deg, d_model)(
      nidx, didx, node_table, degree_table)
  return out.reshape(b, sq, d_model)


# speedup vs baseline: 4.0068x; 1.2771x over previous
import functools

import jax
import jax.numpy as jnp
from jax import lax
from jax.experimental import pallas as pl
from jax.experimental.pallas import tpu as pltpu
from jax.experimental.pallas import tpu_sc as plsc

L = 16
NW = 32
CH = 512


def _make_sc_kernel(n_total, n_deg, d_model):
  assert n_total % (NW * CH) == 0
  n_per_w = n_total // NW
  n_chunks = n_per_w // CH
  mesh = plsc.VectorSubcoreMesh(core_axis_name="c", subcore_axis_name="s")

  @functools.partial(
      pl.kernel,
      out_type=jax.ShapeDtypeStruct((n_total, d_model), jnp.float32),
      mesh=mesh,
      scratch_types=[
          pltpu.VMEM((CH,), jnp.int32),
          pltpu.VMEM((CH,), jnp.int32),
          pltpu.VMEM((CH, d_model), jnp.float32),
          pltpu.VMEM((n_deg, d_model), jnp.float32),
          pltpu.VMEM((d_model,), jnp.float32),
          pltpu.VMEM_SHARED((2 * n_deg, d_model), jnp.float32),
          pltpu.SemaphoreType.DMA,
      ],
      compiler_params=pltpu.CompilerParams(use_tc_tiling_on_sc=False),
  )
  def body(nidx_hbm, didx_hbm, node_hbm, deg_hbm, out_hbm,
           nidx_v, didx_v, rows_v, dt_v, nt0_v, dt_sp, sem):
    c = lax.axis_index("c")
    s = lax.axis_index("s")
    wid = s * 2 + c
    base_w = wid * n_per_w

    @pl.when(s == 0)
    def _stage():
      pltpu.sync_copy(deg_hbm, dt_v)
      pltpu.sync_copy(node_hbm.at[0], nt0_v)
      for i in range(d_model // L):
        dt_v[0, pl.ds(i * L, L)] = jnp.zeros((L,), jnp.float32)
      pltpu.sync_copy(dt_v, dt_sp.at[pl.ds(0, n_deg)])

      def sub_row(r, carry):
        for i in range(d_model // L):
          sl = pl.ds(i * L, L)
          dt_v[r, sl] = dt_v[r, sl] - nt0_v[sl]
        return carry
      lax.fori_loop(0, n_deg, sub_row, 0)
      pltpu.sync_copy(dt_v, dt_sp.at[pl.ds(n_deg, n_deg)])
    plsc.subcore_barrier()

    def chunk(g, carry):
      base = base_w + g * CH
      pltpu.sync_copy(nidx_hbm.at[pl.ds(base, CH)], nidx_v)
      pltpu.sync_copy(didx_hbm.at[pl.ds(base, CH)], didx_v)
      for grp in range(CH // L):
        sl = pl.ds(grp * L, L)
        d = didx_v[sl]
        didx_v[sl] = jnp.where(nidx_v[sl] == 0, d + n_deg, d)
      pltpu.async_copy(node_hbm.at[nidx_v], rows_v, sem).wait()
      pltpu.async_copy(dt_sp.at[didx_v], rows_v, sem, add=True).wait()
      pltpu.sync_copy(rows_v, out_hbm.at[pl.ds(base, CH)])
      return carry

    lax.fori_loop(0, n_chunks, chunk, 0)

  return body


def kernel(item_seq, item_degree, node_table, degree_table):
  b, sq = item_seq.shape
  n_deg, d_model = degree_table.shape
  n_total = b * sq
  nidx = item_seq.reshape(n_total).astype(jnp.int32)
  didx = item_degree.reshape(n_total).astype(jnp.int32)
  out = _make_sc_kernel(n_total, n_deg, d_model)(
      nidx, didx, node_table, degree_table)
  return out.reshape(b, sq, d_model)


# speedup vs baseline: 4.7460x; 1.1845x over previous
import functools

import jax
import jax.numpy as jnp
from jax import lax
from jax.experimental import pallas as pl
from jax.experimental.pallas import tpu as pltpu
from jax.experimental.pallas import tpu_sc as plsc

L = 16
NW = 32
CH = 512


def _make_sc_kernel(n_total, n_deg, d_model):
  assert n_total % (NW * CH) == 0
  n_per_w = n_total // NW
  n_chunks = n_per_w // CH
  assert n_chunks % 2 == 0 and n_chunks >= 4
  assert CH >= n_deg
  mesh = plsc.VectorSubcoreMesh(core_axis_name="c", subcore_axis_name="s")

  @functools.partial(
      pl.kernel,
      out_type=jax.ShapeDtypeStruct((n_total, d_model), jnp.float32),
      mesh=mesh,
      scratch_types=[
          pltpu.VMEM((CH,), jnp.int32),
          pltpu.VMEM((CH,), jnp.int32),
          pltpu.VMEM((CH,), jnp.int32),
          pltpu.VMEM((CH,), jnp.int32),
          pltpu.VMEM((CH, d_model), jnp.float32),
          pltpu.VMEM((CH, d_model), jnp.float32),
          pltpu.VMEM((d_model,), jnp.float32),
          pltpu.VMEM_SHARED((2 * n_deg, d_model), jnp.float32),
          pltpu.SemaphoreType.DMA,
          pltpu.SemaphoreType.DMA,
          pltpu.SemaphoreType.DMA,
          pltpu.SemaphoreType.DMA,
          pltpu.SemaphoreType.DMA,
          pltpu.SemaphoreType.DMA,
          pltpu.SemaphoreType.DMA,
      ],
      compiler_params=pltpu.CompilerParams(use_tc_tiling_on_sc=False),
  )
  def body(nidx_hbm, didx_hbm, node_hbm, deg_hbm, out_hbm,
           nidx0, nidx1, didx0, didx1, rows0, rows1, nt0_v, dt_sp,
           isem0, isem1, gsem0, gsem1, asem, osem0, osem1):
    c = lax.axis_index("c")
    s = lax.axis_index("s")
    wid = s * 2 + c
    base_w = wid * n_per_w
    nidx = [nidx0, nidx1]
    didx = [didx0, didx1]
    rows = [rows0, rows1]
    isem = [isem0, isem1]
    gsem = [gsem0, gsem1]
    osem = [osem0, osem1]

    @pl.when(s == 0)
    def _stage():
      pltpu.sync_copy(deg_hbm, rows0.at[pl.ds(0, n_deg)])
      pltpu.sync_copy(node_hbm.at[0], nt0_v)
      for i in range(d_model // L):
        rows0[0, pl.ds(i * L, L)] = jnp.zeros((L,), jnp.float32)
      pltpu.sync_copy(rows0.at[pl.ds(0, n_deg)], dt_sp.at[pl.ds(0, n_deg)])

      def sub_row(r, carry):
        for i in range(d_model // L):
          sl = pl.ds(i * L, L)
          rows0[r, sl] = rows0[r, sl] - nt0_v[sl]
        return carry
      lax.fori_loop(0, n_deg, sub_row, 0)
      pltpu.sync_copy(rows0.at[pl.ds(0, n_deg)], dt_sp.at[pl.ds(n_deg, n_deg)])
    plsc.subcore_barrier()

    def issue_idx(g, b):
      base = base_w + g * CH
      pltpu.async_copy(nidx_hbm.at[pl.ds(base, CH)], nidx[b], isem[b])
      pltpu.async_copy(didx_hbm.at[pl.ds(base, CH)], didx[b], isem[b])

    def wait_idx(g, b):
      base = base_w + g * CH
      pltpu.make_async_copy(nidx_hbm.at[pl.ds(base, CH)], nidx[b],
                            isem[b]).wait()
      pltpu.make_async_copy(didx_hbm.at[pl.ds(base, CH)], didx[b],
                            isem[b]).wait()

    def remap(b):
      for grp in range(CH // L):
        sl = pl.ds(grp * L, L)
        d = didx[b][sl]
        didx[b][sl] = jnp.where(nidx[b][sl] == 0, d + n_deg, d)

    def out_slice(g):
      return out_hbm.at[pl.ds(base_w + g * CH, CH)]

    def step(g, b, wait_out_prev2, idx_guarded):
      b1 = 1 - b
      acp = pltpu.async_copy(dt_sp.at[didx[b1]], rows[b1], asem, add=True)
      wait_idx(g, b)
      remap(b)
      if wait_out_prev2:
        pltpu.make_async_copy(rows[b], out_slice(g - 2), osem[b]).wait()
      gcp = pltpu.async_copy(node_hbm.at[nidx[b]], rows[b], gsem[b])
      acp.wait()
      pltpu.async_copy(rows[b1], out_slice(g - 1), osem[b1])
      if idx_guarded:
        @pl.when(g + 1 < n_chunks)
        def _pf():
          issue_idx(g + 1, b1)
      else:
        issue_idx(g + 1, b1)
      gcp.wait()

    issue_idx(0, 0)
    issue_idx(1, 1)
    wait_idx(0, 0)
    remap(0)
    pltpu.async_copy(node_hbm.at[nidx[0]], rows[0], gsem[0]).wait()
    step(1, 1, wait_out_prev2=False, idx_guarded=False)

    def loop_body(t, carry):
      step(2 * t + 2, 0, wait_out_prev2=True, idx_guarded=False)
      step(2 * t + 3, 1, wait_out_prev2=True, idx_guarded=True)
      return carry
    lax.fori_loop(0, n_chunks // 2 - 1, loop_body, 0)

    last = n_chunks - 1
    lb = last % 2
    acp = pltpu.async_copy(dt_sp.at[didx[lb]], rows[lb], asem, add=True)
    acp.wait()
    ocp = pltpu.async_copy(rows[lb], out_slice(last), osem[lb])
    pltpu.make_async_copy(rows[1 - lb], out_slice(last - 1),
                          osem[1 - lb]).wait()
    ocp.wait()

  return body


def kernel(item_seq, item_degree, node_table, degree_table):
  b, sq = item_seq.shape
  n_deg, d_model = degree_table.shape
  n_total = b * sq
  nidx = item_seq.reshape(n_total).astype(jnp.int32)
  didx = item_degree.reshape(n_total).astype(jnp.int32)
  out = _make_sc_kernel(n_total, n_deg, d_model)(
      nidx, didx, node_table, degree_table)
  return out.reshape(b, sq, d_model)


# speedup vs baseline: 4.7774x; 1.0066x over previous
import functools

import jax
import jax.numpy as jnp
from jax import lax
from jax.experimental import pallas as pl
from jax.experimental.pallas import tpu as pltpu
from jax.experimental.pallas import tpu_sc as plsc

L = 16
NW = 32
BR = 4


def _make_sc_kernel(n_batch, seq, n_deg, d_model):
  ch = BR * seq
  assert n_batch % (NW * BR) == 0
  br_per_w = n_batch // NW
  n_per_w = br_per_w * seq
  n_chunks = br_per_w // BR
  assert n_chunks % 2 == 0 and n_chunks >= 4
  assert ch >= n_deg
  assert ch % L == 0
  mesh = plsc.VectorSubcoreMesh(core_axis_name="c", subcore_axis_name="s")

  @functools.partial(
      pl.kernel,
      out_type=jax.ShapeDtypeStruct((n_batch, seq, d_model), jnp.float32),
      mesh=mesh,
      scratch_types=[
          pltpu.VMEM((ch,), jnp.int32),
          pltpu.VMEM((ch,), jnp.int32),
          pltpu.VMEM((ch,), jnp.int32),
          pltpu.VMEM((ch,), jnp.int32),
          pltpu.VMEM((ch, d_model), jnp.float32),
          pltpu.VMEM((ch, d_model), jnp.float32),
          pltpu.VMEM((d_model,), jnp.float32),
          pltpu.VMEM_SHARED((2 * n_deg, d_model), jnp.float32),
          pltpu.SemaphoreType.DMA,
          pltpu.SemaphoreType.DMA,
          pltpu.SemaphoreType.DMA,
          pltpu.SemaphoreType.DMA,
          pltpu.SemaphoreType.DMA,
          pltpu.SemaphoreType.DMA,
          pltpu.SemaphoreType.DMA,
      ],
      compiler_params=pltpu.CompilerParams(use_tc_tiling_on_sc=False),
  )
  def body(nidx_hbm, didx_hbm, node_hbm, deg_hbm, out_hbm,
           nidx0, nidx1, didx0, didx1, rows0, rows1, nt0_v, dt_sp,
           isem0, isem1, gsem0, gsem1, asem, osem0, osem1):
    c = lax.axis_index("c")
    s = lax.axis_index("s")
    wid = s * 2 + c
    base_w = wid * n_per_w
    brow_w = wid * br_per_w
    nidx = [nidx0, nidx1]
    didx = [didx0, didx1]
    rows = [rows0, rows1]
    isem = [isem0, isem1]
    gsem = [gsem0, gsem1]
    osem = [osem0, osem1]

    @pl.when(s == 0)
    def _stage():
      pltpu.sync_copy(deg_hbm, rows0.at[pl.ds(0, n_deg)])
      pltpu.sync_copy(node_hbm.at[0], nt0_v)
      for i in range(d_model // L):
        rows0[0, pl.ds(i * L, L)] = jnp.zeros((L,), jnp.float32)
      pltpu.sync_copy(rows0.at[pl.ds(0, n_deg)], dt_sp.at[pl.ds(0, n_deg)])

      def sub_row(r, carry):
        for i in range(d_model // L):
          sl = pl.ds(i * L, L)
          rows0[r, sl] = rows0[r, sl] - nt0_v[sl]
        return carry
      lax.fori_loop(0, n_deg, sub_row, 0)
      pltpu.sync_copy(rows0.at[pl.ds(0, n_deg)], dt_sp.at[pl.ds(n_deg, n_deg)])
    plsc.subcore_barrier()

    def issue_idx(g, b):
      base = base_w + g * ch
      pltpu.async_copy(nidx_hbm.at[pl.ds(base, ch)], nidx[b], isem[b])
      pltpu.async_copy(didx_hbm.at[pl.ds(base, ch)], didx[b], isem[b])

    def wait_idx(g, b):
      base = base_w + g * ch
      pltpu.make_async_copy(nidx_hbm.at[pl.ds(base, ch)], nidx[b],
                            isem[b]).wait()
      pltpu.make_async_copy(didx_hbm.at[pl.ds(base, ch)], didx[b],
                            isem[b]).wait()

    def remap(b):
      for grp in range(ch // L):
        sl = pl.ds(grp * L, L)
        d = didx[b][sl]
        didx[b][sl] = jnp.where(nidx[b][sl] == 0, d + n_deg, d)

    def write_out(g, b):
      for j in range(BR):
        pltpu.async_copy(rows[b].at[pl.ds(j * seq, seq)],
                         out_hbm.at[brow_w + g * BR + j], osem[b])

    def wait_out(g, b):
      for j in range(BR):
        pltpu.make_async_copy(rows[b].at[pl.ds(j * seq, seq)],
                              out_hbm.at[brow_w + g * BR + j],
                              osem[b]).wait()

    def step(g, b, wait_out_prev2, idx_guarded):
      b1 = 1 - b
      acp = pltpu.async_copy(dt_sp.at[didx[b1]], rows[b1], asem, add=True)
      wait_idx(g, b)
      remap(b)
      if wait_out_prev2:
        wait_out(g - 2, b)
      gcp = pltpu.async_copy(node_hbm.at[nidx[b]], rows[b], gsem[b])
      acp.wait()
      write_out(g - 1, b1)
      if idx_guarded:
        @pl.when(g + 1 < n_chunks)
        def _pf():
          issue_idx(g + 1, b1)
      else:
        issue_idx(g + 1, b1)
      gcp.wait()

    issue_idx(0, 0)
    issue_idx(1, 1)
    wait_idx(0, 0)
    remap(0)
    pltpu.async_copy(node_hbm.at[nidx[0]], rows[0], gsem[0]).wait()
    step(1, 1, wait_out_prev2=False, idx_guarded=False)

    def loop_body(t, carry):
      step(2 * t + 2, 0, wait_out_prev2=True, idx_guarded=False)
      step(2 * t + 3, 1, wait_out_prev2=True, idx_guarded=True)
      return carry
    lax.fori_loop(0, n_chunks // 2 - 1, loop_body, 0)

    last = n_chunks - 1
    lb = last % 2
    acp = pltpu.async_copy(dt_sp.at[didx[lb]], rows[lb], asem, add=True)
    acp.wait()
    write_out(last, lb)
    wait_out(last - 1, 1 - lb)
    wait_out(last, lb)

  return body


def kernel(item_seq, item_degree, node_table, degree_table):
  b, sq = item_seq.shape
  n_deg, d_model = degree_table.shape
  n_total = b * sq
  nidx = item_seq.reshape(n_total).astype(jnp.int32)
  didx = item_degree.reshape(n_total).astype(jnp.int32)
  return _make_sc_kernel(b, sq, n_deg, d_model)(
      nidx, didx, node_table, degree_table)


# speedup vs baseline: 7.8232x; 1.6375x over previous
import functools

import jax
import jax.numpy as jnp
from jax import lax
from jax.experimental import pallas as pl
from jax.experimental.pallas import tpu as pltpu
from jax.experimental.pallas import tpu_sc as plsc

L = 16
NW = 32
BR = 4


def _make_sc_kernel(n_batch, seq, n_deg, d_model):
  ch = BR * seq
  assert n_batch % (NW * BR) == 0
  br_per_w = n_batch // NW
  n_per_w = br_per_w * seq
  n_chunks = br_per_w // BR
  assert n_chunks % 2 == 0 and n_chunks >= 4
  assert ch >= n_deg
  assert ch % L == 0
  mesh = plsc.VectorSubcoreMesh(core_axis_name="c", subcore_axis_name="s")

  @functools.partial(
      pl.kernel,
      out_type=jax.ShapeDtypeStruct((n_batch, seq, 2 * d_model), jnp.float32),
      mesh=mesh,
      scratch_types=[
          pltpu.VMEM((ch,), jnp.int32),
          pltpu.VMEM((ch,), jnp.int32),
          pltpu.VMEM((ch,), jnp.int32),
          pltpu.VMEM((ch,), jnp.int32),
          pltpu.VMEM((ch, d_model), jnp.float32),
          pltpu.VMEM((ch, d_model), jnp.float32),
          pltpu.VMEM((d_model,), jnp.float32),
          pltpu.VMEM_SHARED((2 * n_deg, d_model), jnp.float32),
          pltpu.SemaphoreType.DMA,
          pltpu.SemaphoreType.DMA,
          pltpu.SemaphoreType.DMA,
          pltpu.SemaphoreType.DMA,
          pltpu.SemaphoreType.DMA,
          pltpu.SemaphoreType.DMA,
          pltpu.SemaphoreType.DMA,
      ],
      compiler_params=pltpu.CompilerParams(use_tc_tiling_on_sc=False),
  )
  def body(nidx_hbm, didx_hbm, node_hbm, deg_hbm, out_hbm,
           nidx0, nidx1, didx0, didx1, rows0, rows1, nt0_v, dt_sp,
           isem0, isem1, gsem0, gsem1, asem, osem0, osem1):
    c = lax.axis_index("c")
    s = lax.axis_index("s")
    wid = s * 2 + c
    base_w = wid * n_per_w
    brow_w = wid * br_per_w
    nidx = [nidx0, nidx1]
    didx = [didx0, didx1]
    rows = [rows0, rows1]
    isem = [isem0, isem1]
    gsem = [gsem0, gsem1]
    osem = [osem0, osem1]

    @pl.when(s == 0)
    def _stage():
      pltpu.sync_copy(deg_hbm, rows0.at[pl.ds(0, n_deg)])
      pltpu.sync_copy(node_hbm.at[0], nt0_v)
      for i in range(d_model // L):
        rows0[0, pl.ds(i * L, L)] = jnp.zeros((L,), jnp.float32)
      pltpu.sync_copy(rows0.at[pl.ds(0, n_deg)], dt_sp.at[pl.ds(0, n_deg)])

      def sub_row(r, carry):
        for i in range(d_model // L):
          sl = pl.ds(i * L, L)
          rows0[r, sl] = rows0[r, sl] - nt0_v[sl]
        return carry
      lax.fori_loop(0, n_deg, sub_row, 0)
      pltpu.sync_copy(rows0.at[pl.ds(0, n_deg)], dt_sp.at[pl.ds(n_deg, n_deg)])
    plsc.subcore_barrier()

    def issue_idx(g, b):
      base = base_w + g * ch
      pltpu.async_copy(nidx_hbm.at[pl.ds(base, ch)], nidx[b], isem[b])
      pltpu.async_copy(didx_hbm.at[pl.ds(base, ch)], didx[b], isem[b])

    def wait_idx(g, b):
      base = base_w + g * ch
      pltpu.make_async_copy(nidx_hbm.at[pl.ds(base, ch)], nidx[b],
                            isem[b]).wait()
      pltpu.make_async_copy(didx_hbm.at[pl.ds(base, ch)], didx[b],
                            isem[b]).wait()

    def remap(b):
      for grp in range(ch // L):
        sl = pl.ds(grp * L, L)
        d = didx[b][sl]
        didx[b][sl] = jnp.where(nidx[b][sl] == 0, d + n_deg, d)

    def write_out(g, b):
      for j in range(BR):
        pltpu.async_copy(rows[b].at[pl.ds(j * seq, seq)],
                         out_hbm.at[brow_w + g * BR + j, slice(None),
                                    pl.ds(0, d_model)], osem[b])

    def wait_out(g, b):
      for j in range(BR):
        pltpu.make_async_copy(rows[b].at[pl.ds(j * seq, seq)],
                              out_hbm.at[brow_w + g * BR + j, slice(None),
                                         pl.ds(0, d_model)],
                              osem[b]).wait()

    def step(g, b, wait_out_prev2, idx_guarded):
      b1 = 1 - b
      acp = pltpu.async_copy(dt_sp.at[didx[b1]], rows[b1], asem, add=True)
      wait_idx(g, b)
      remap(b)
      if wait_out_prev2:
        wait_out(g - 2, b)
      gcp = pltpu.async_copy(node_hbm.at[nidx[b]], rows[b], gsem[b])
      acp.wait()
      write_out(g - 1, b1)
      if idx_guarded:
        @pl.when(g + 1 < n_chunks)
        def _pf():
          issue_idx(g + 1, b1)
      else:
        issue_idx(g + 1, b1)
      gcp.wait()

    issue_idx(0, 0)
    issue_idx(1, 1)
    wait_idx(0, 0)
    remap(0)
    pltpu.async_copy(node_hbm.at[nidx[0]], rows[0], gsem[0]).wait()
    step(1, 1, wait_out_prev2=False, idx_guarded=False)

    def loop_body(t, carry):
      step(2 * t + 2, 0, wait_out_prev2=True, idx_guarded=False)
      step(2 * t + 3, 1, wait_out_prev2=True, idx_guarded=True)
      return carry
    lax.fori_loop(0, n_chunks // 2 - 1, loop_body, 0)

    last = n_chunks - 1
    lb = last % 2
    acp = pltpu.async_copy(dt_sp.at[didx[lb]], rows[lb], asem, add=True)
    acp.wait()
    write_out(last, lb)
    wait_out(last - 1, 1 - lb)
    wait_out(last, lb)

  return body


def kernel(item_seq, item_degree, node_table, degree_table):
  b, sq = item_seq.shape
  n_deg, d_model = degree_table.shape
  n_total = b * sq
  nidx = item_seq.reshape(n_total).astype(jnp.int32)
  didx = item_degree.reshape(n_total).astype(jnp.int32)
  res = _make_sc_kernel(b, sq, n_deg, d_model)(
      nidx, didx, node_table, degree_table)
  return res[:, :, :d_model]
